# baseline (speedup 1.0000x reference)
import functools

import jax
import jax.numpy as jnp
from jax import lax
from jax.experimental import pallas as pl
from jax.experimental.pallas import tpu as pltpu
from jax.experimental.pallas import tpu_sc as plsc

F32 = jnp.float32
BN_EPS = 1e-5
NC, NS, LANES = 2, 16, 16
NW = NC * NS


def _block_rows(rows, cap=2048):
    best = 8
    for b in range(8, cap + 1, 8):
        if rows % b == 0:
            best = b
    return best


def _pq_body(x_ref, wa_ref, wb_ref, p_ref, q_ref):
    x = x_ref[...]
    p_ref[...] = jnp.dot(x, wa_ref[...], preferred_element_type=F32)
    q_ref[...] = jnp.dot(x, wb_ref[...], preferred_element_type=F32)


def _compute_pq(x, wa, wb):
    n, l = x.shape
    br = _block_rows(n)
    return pl.pallas_call(
        _pq_body,
        grid=(n // br,),
        in_specs=[
            pl.BlockSpec((br, l), lambda i: (i, 0)),
            pl.BlockSpec((l, l), lambda i: (0, 0)),
            pl.BlockSpec((l, l), lambda i: (0, 0)),
        ],
        out_specs=[
            pl.BlockSpec((br, l), lambda i: (i, 0)),
            pl.BlockSpec((br, l), lambda i: (i, 0)),
        ],
        out_shape=[jax.ShapeDtypeStruct((n, l), F32)] * 2,
    )(x, wa, wb)


def _stats_accumulate(h, stats_ref):
    @pl.when(pl.program_id(0) == 0)
    def _():
        stats_ref[...] = jnp.zeros_like(stats_ref)

    stats_ref[0:1, :] += jnp.sum(h, axis=0, keepdims=True)
    stats_ref[1:2, :] += jnp.sum(h * h, axis=0, keepdims=True)


def _edge_pass1_body(g_ref, ea_ref, wc_ref, stats_ref):
    hb = g_ref[...].astype(F32) + jnp.dot(ea_ref[...], wc_ref[...],
                                          preferred_element_type=F32)
    _stats_accumulate(hb, stats_ref)


def _edge_pass1(g, ea, wc):
    e, l = g.shape
    br = _block_rows(e)
    return pl.pallas_call(
        _edge_pass1_body,
        grid=(e // br,),
        in_specs=[
            pl.BlockSpec((br, l), lambda i: (i, 0)),
            pl.BlockSpec((br, l), lambda i: (i, 0)),
            pl.BlockSpec((l, l), lambda i: (0, 0)),
        ],
        out_specs=pl.BlockSpec((2, l), lambda i: (0, 0)),
        out_shape=jax.ShapeDtypeStruct((2, l), F32),
        compiler_params=pltpu.CompilerParams(
            dimension_semantics=("arbitrary",)),
    )(g, ea, wc)


def _edge_pass2_body(g_ref, ea_ref, wc_ref, scale_ref, shift_ref, w2_ref,
                     b2_ref, out_ref):
    ea = ea_ref[...]
    hb = g_ref[...].astype(F32) + jnp.dot(ea, wc_ref[...],
                                          preferred_element_type=F32)
    hn = jnp.maximum(hb * scale_ref[...] + shift_ref[...], 0.0)
    y = jnp.dot(hn, w2_ref[...], preferred_element_type=F32) + b2_ref[...]
    out_ref[...] = jnp.maximum(y, 0.0) + ea


def _edge_pass2(g, ea, wc, scale, shift, w2, b2):
    e, l = g.shape
    br = _block_rows(e)
    row = pl.BlockSpec((br, l), lambda i: (i, 0))
    vec = pl.BlockSpec((1, l), lambda i: (0, 0))
    w = pl.BlockSpec((l, l), lambda i: (0, 0))
    return pl.pallas_call(
        _edge_pass2_body,
        grid=(e // br,),
        in_specs=[row, row, w, vec, vec, w, vec],
        out_specs=row,
        out_shape=jax.ShapeDtypeStruct((e, l), F32),
    )(g, ea, wc, scale, shift, w2, b2.reshape(1, l))


def _node_pass1_body(x_ref, a0_ref, a1_ref, wa_ref, wb_ref, b_ref,
                     h_ref, stats_ref):
    h = (jnp.dot(x_ref[...], wa_ref[...], preferred_element_type=F32)
         + jnp.dot(a0_ref[...] + a1_ref[...], wb_ref[...],
                   preferred_element_type=F32)
         + b_ref[...])
    h_ref[...] = h
    _stats_accumulate(h, stats_ref)


def _node_pass1(x, a0, a1, wa, wb, b1):
    n, l = x.shape
    br = _block_rows(n)
    row = pl.BlockSpec((br, l), lambda i: (i, 0))
    w = pl.BlockSpec((l, l), lambda i: (0, 0))
    return pl.pallas_call(
        _node_pass1_body,
        grid=(n // br,),
        in_specs=[row, row, row, w, w, pl.BlockSpec((1, l), lambda i: (0, 0))],
        out_specs=[row, pl.BlockSpec((2, l), lambda i: (0, 0))],
        out_shape=[jax.ShapeDtypeStruct((n, l), F32),
                   jax.ShapeDtypeStruct((2, l), F32)],
        compiler_params=pltpu.CompilerParams(
            dimension_semantics=("arbitrary",)),
    )(x, a0, a1, wa, wb, b1.reshape(1, l))


def _node_pass2_body(h_ref, res_ref, scale_ref, shift_ref, w2_ref, b2_ref,
                     out_ref):
    hn = jnp.maximum(h_ref[...] * scale_ref[...] + shift_ref[...], 0.0)
    y = jnp.dot(hn, w2_ref[...], preferred_element_type=F32) + b2_ref[...]
    out_ref[...] = jnp.maximum(y, 0.0) + res_ref[...]


def _node_pass2(h, res, scale, shift, w2, b2):
    r, l = h.shape
    br = _block_rows(r)
    row = pl.BlockSpec((br, l), lambda i: (i, 0))
    vec = pl.BlockSpec((1, l), lambda i: (0, 0))
    return pl.pallas_call(
        _node_pass2_body,
        grid=(r // br,),
        in_specs=[row, row, vec, vec,
                  pl.BlockSpec((l, l), lambda i: (0, 0)), vec],
        out_specs=row,
        out_shape=jax.ShapeDtypeStruct((r, l), F32),
    )(h, res, scale, shift, w2, b2.reshape(1, l))


def _bn_affine(stats, count, g1, be1, b1=None):
    mean = stats[0] / count
    var = stats[1] / count - mean * mean
    if b1 is not None:
        mean = mean + b1
    inv = g1 * lax.rsqrt(var + BN_EPS)
    scale = inv.reshape(1, -1)
    if b1 is not None:
        shift = (be1 + (b1 - mean) * inv).reshape(1, -1)
    else:
        shift = (be1 - mean * inv).reshape(1, -1)
    return scale, shift


_CG = 80


def _make_gather_add(n, e, l):
    nper = e // NW
    lw = l // 2
    assert nper % _CG == 0 and nper % 16 == 0
    nchunks = nper // _CG
    npairs = nchunks // 2
    has_tail = nchunks % 2 == 1
    assert npairs >= 2
    assert _CG % 16 == 0
    mesh = plsc.VectorSubcoreMesh(core_axis_name="c", subcore_axis_name="s",
                                  num_cores=NC, num_subcores=NS)

    @functools.partial(
        pl.kernel,
        out_type=jax.ShapeDtypeStruct((e, l), F32),
        mesh=mesh,
        compiler_params=pltpu.CompilerParams(needs_layout_passes=False,
                                             use_tc_tiling_on_sc=False),
        scratch_types=[
            pltpu.VMEM((nchunks, _CG), jnp.int32),
            pltpu.VMEM((nchunks, _CG), jnp.int32),
            pltpu.VMEM((_CG, lw), jnp.int32),
            pltpu.VMEM((_CG, lw), jnp.int32),
            pltpu.VMEM((_CG, l), F32),
            pltpu.VMEM((_CG, lw), jnp.int32),
            pltpu.VMEM((_CG, lw), jnp.int32),
            pltpu.VMEM((_CG, l), F32),
        ] + [pltpu.SemaphoreType.DMA] * 4,
    )
    def gather_add(p_hbm, q_hbm, dsts_hbm, srcs_hbm, g_hbm,
                   idxd, idxs, prow0, qrow0, orow0, prow1, qrow1, orow1,
                   semg0, semg1, semw0, semw1):
        wid = lax.axis_index("s") * NC + lax.axis_index("c")
        base0 = wid * nper
        pltpu.sync_copy(dsts_hbm.at[wid], idxd)
        pltpu.sync_copy(srcs_hbm.at[wid], idxs)

        bufs = ((prow0, qrow0, orow0, semg0, semw0),
                (prow1, qrow1, orow1, semg1, semw1))

        def fire_gather(c, b):
            prow, qrow, _, semg, _ = bufs[b]
            pltpu.async_copy(p_hbm.at[idxd.at[c]], prow, semg)
            pltpu.async_copy(q_hbm.at[idxs.at[c]], qrow, semg)

        def wait_gather(b):
            prow, qrow, _, semg, _ = bufs[b]
            pltpu.make_async_copy(p_hbm.at[idxd.at[0]], prow, semg).wait()
            pltpu.make_async_copy(q_hbm.at[idxs.at[0]], qrow, semg).wait()

        def add_rows(b):
            prow, qrow, orow, _, _ = bufs[b]

            @plsc.parallel_loop(0, _CG, 2, unroll=2)
            def _(r):
                for rr in range(2):
                    for j in range(lw // LANES):
                        sl = pl.ds(j * LANES, LANES)
                        x = plsc.bitcast(prow[r + rr, sl], jnp.bfloat16)
                        y = plsc.bitcast(qrow[r + rr, sl], jnp.bfloat16)
                        a, b2 = plsc.unpack(x + y,
                                            format=plsc.PackFormat.INTERLEAVED)
                        base = j * 2 * LANES
                        orow[r + rr, pl.ds(base, LANES)] = a
                        orow[r + rr, pl.ds(base + LANES, LANES)] = b2

        def fire_wb(c, b):
            _, _, orow, _, semw = bufs[b]
            base = pl.multiple_of(base0 + c * _CG, 16)
            pltpu.async_copy(orow, g_hbm.at[pl.ds(base, _CG)], semw)

        def wait_wb(b):
            _, _, orow, _, semw = bufs[b]
            pltpu.make_async_copy(
                orow, g_hbm.at[pl.ds(base0, _CG)], semw).wait()

        fire_gather(0, 0)
        fire_gather(1, 1)
        wait_gather(0)
        add_rows(0)
        fire_gather(2, 0)
        fire_wb(0, 0)
        wait_gather(1)
        add_rows(1)
        fire_gather(3, 1)
        fire_wb(1, 1)

        def body(i, carry):
            for b in range(2):
                c = 2 * i + b
                wait_gather(b)
                wait_wb(b)
                add_rows(b)

                @pl.when(c + 2 < nchunks)
                def _():
                    fire_gather(c + 2, b)

                fire_wb(c, b)
            return carry

        lax.fori_loop(1, npairs, body, 0)

        if has_tail:
            wait_gather(0)
            wait_wb(0)
            add_rows(0)
            fire_wb(nchunks - 1, 0)
        wait_wb(0)
        wait_wb(1)

    return gather_add


def _make_scatter_sum(n, e, l):
    nper = e // NW
    nchunks = nper // _CG
    npairs = nchunks // 2
    has_tail = nchunks % 2 == 1
    zr = 64
    npad = -(-n // (NS * 128)) * NS * 128
    rpt = npad // NS
    tail_start = (n // rpt) * rpt
    tail_len = n - tail_start
    assert n % 8 == 0
    mesh = plsc.VectorSubcoreMesh(core_axis_name="c", subcore_axis_name="s",
                                  num_cores=NC, num_subcores=NS)

    @functools.partial(
        pl.kernel,
        out_type=[jax.ShapeDtypeStruct((n, l), F32)] * NC,
        mesh=mesh,
        scratch_types=[
            pltpu.VMEM((nchunks, _CG), jnp.int32),
            pltpu.VMEM((_CG, l), F32),
            pltpu.VMEM((_CG, l), F32),
            pltpu.VMEM((zr, l), F32),
            pltpu.VMEM_SHARED((npad, l), F32),
        ] + [pltpu.SemaphoreType.DMA] * 4,
    )
    def scatter_sum(upd_hbm, dsts_hbm, o0, o1,
                    idx_v, rows0, rows1, zeros_v, acc,
                    seml0, seml1, semsc0, semsc1):
        cid = lax.axis_index("c")
        sid = lax.axis_index("s")

        def zrow(r, c2):
            for j in range(l // LANES):
                zeros_v[r, pl.ds(j * LANES, LANES)] = jnp.zeros((LANES,), F32)
            return c2

        lax.fori_loop(0, zr, zrow, 0)
        row0 = sid * rpt
        for k in range(rpt // zr):
            pltpu.sync_copy(zeros_v, acc.at[pl.ds(row0 + k * zr, zr)])
        plsc.subcore_barrier()

        wid = cid * NS + sid
        base0 = wid * nper
        pltpu.sync_copy(dsts_hbm.at[wid], idx_v)

        bufs = ((rows0, seml0, semsc0), (rows1, seml1, semsc1))

        def fire_load(c, b):
            rows, seml, _ = bufs[b]
            base = pl.multiple_of(base0 + c * _CG, 8)
            pltpu.async_copy(upd_hbm.at[pl.ds(base, _CG)], rows, seml)

        def wait_load(b):
            rows, seml, _ = bufs[b]
            pltpu.make_async_copy(
                upd_hbm.at[pl.ds(base0, _CG)], rows, seml).wait()

        def fire_scat(c, b):
            rows, _, semsc = bufs[b]
            pltpu.async_copy(rows, acc.at[idx_v.at[c]], semsc, add=True)

        def wait_scat(b):
            rows, _, semsc = bufs[b]
            pltpu.make_async_copy(rows, acc.at[idx_v.at[0]], semsc).wait()

        fire_load(0, 0)
        fire_load(1, 1)

        def body(i, carry):
            for b in range(2):
                wait_load(b)
                fire_scat(2 * i + b, b)
            for b in range(2):
                c = 2 * i + b
                wait_scat(b)

                @pl.when(c + 2 < nchunks)
                def _():
                    fire_load(c + 2, b)
            return carry

        lax.fori_loop(0, npairs, body, 0)

        if has_tail:
            wait_load(0)
            fire_scat(nchunks - 1, 0)
            wait_scat(0)
        plsc.subcore_barrier()

        def copy_out(out_ref):
            @pl.when(row0 + rpt <= n)
            def _():
                pltpu.sync_copy(acc.at[pl.ds(row0, rpt)],
                                out_ref.at[pl.ds(row0, rpt)])
            if tail_len > 0:
                @pl.when(row0 == tail_start)
                def _():
                    pltpu.sync_copy(acc.at[pl.ds(tail_start, tail_len)],
                                    out_ref.at[pl.ds(tail_start, tail_len)])

        @pl.when(cid == 0)
        def _():
            copy_out(o0)

        @pl.when(cid == 1)
        def _():
            copy_out(o1)

    return scatter_sum


def _pack_bf16_pairs(t):
    n, l = t.shape
    tb = t.astype(jnp.bfloat16).reshape(n, l // 2, 2)
    return jax.lax.bitcast_convert_type(tb, jnp.int32)


def _unpack_perm(l):
    perm = []
    for j in range(l // (2 * LANES)):
        base = j * 2 * LANES
        perm += [base + 2 * i for i in range(LANES)]
        perm += [base + 2 * i + 1 for i in range(LANES)]
    return jnp.asarray(perm, jnp.int32)


def kernel(x, edge_attr, params, edge_index):
    n, l = x.shape
    e = edge_attr.shape[0]
    nper = e // NW
    src = edge_index[0].reshape(NW, nper // _CG, _CG)
    dst = edge_index[1].reshape(NW, nper // _CG, _CG)

    gather_add = _make_gather_add(n, e, l)
    scatter_sum = _make_scatter_sum(n, e, l)

    perm = _unpack_perm(l)
    for layer in params:
        ep, npar = layer['edge'], layer['node']
        wa, wb = ep['W1'][:l], ep['W1'][l:2 * l]
        wc = ep['W1'][2 * l:][:, perm]
        p, q = _compute_pq(x, wa, wb)
        g = gather_add(_pack_bf16_pairs(p), _pack_bf16_pairs(q), dst, src)
        stats = _edge_pass1(g, edge_attr, wc)
        scale, shift = _bn_affine(stats, e, ep['g1'][perm], ep['be1'][perm],
                                  ep['b1'][perm])
        upd = _edge_pass2(g, edge_attr, wc, scale, shift, ep['W2'][perm, :],
                          ep['b2'])

        a0, a1 = scatter_sum(upd, dst)

        hn, nstats = _node_pass1(x, a0, a1, npar['W1'][:l], npar['W1'][l:],
                                 npar['b1'])
        nscale, nshift = _bn_affine(nstats, n, npar['g1'], npar['be1'])
        x = _node_pass2(hn, x, nscale, nshift, npar['W2'], npar['b2'])
        edge_attr = upd

    return x, edge_attr


# speedup vs baseline: 1.0632x; 1.0632x over previous
import functools

import jax
import jax.numpy as jnp
from jax import lax
from jax.experimental import pallas as pl
from jax.experimental.pallas import tpu as pltpu
from jax.experimental.pallas import tpu_sc as plsc

F32 = jnp.float32
BN_EPS = 1e-5
NC, NS, LANES = 2, 16, 16
NW = NC * NS


def _block_rows(rows, cap=2048):
    best = 8
    for b in range(8, cap + 1, 8):
        if rows % b == 0:
            best = b
    return best


def _pq_body(x_ref, wa_ref, wb_ref, p_ref, q_ref):
    x = x_ref[...]
    p_ref[...] = jnp.dot(x, wa_ref[...],
                         preferred_element_type=F32).astype(jnp.bfloat16)
    q_ref[...] = jnp.dot(x, wb_ref[...],
                         preferred_element_type=F32).astype(jnp.bfloat16)


def _compute_pq(x, wa, wb):
    n, l = x.shape
    br = _block_rows(n)
    return pl.pallas_call(
        _pq_body,
        grid=(n // br,),
        in_specs=[
            pl.BlockSpec((br, l), lambda i: (i, 0)),
            pl.BlockSpec((l, l), lambda i: (0, 0)),
            pl.BlockSpec((l, l), lambda i: (0, 0)),
        ],
        out_specs=[
            pl.BlockSpec((br, l), lambda i: (i, 0)),
            pl.BlockSpec((br, l), lambda i: (i, 0)),
        ],
        out_shape=[jax.ShapeDtypeStruct((n, l), jnp.bfloat16)] * 2,
    )(x, wa, wb)


def _stats_accumulate(h, stats_ref):
    @pl.when(pl.program_id(0) == 0)
    def _():
        stats_ref[...] = jnp.zeros_like(stats_ref)

    stats_ref[0:1, :] += jnp.sum(h, axis=0, keepdims=True)
    stats_ref[1:2, :] += jnp.sum(h * h, axis=0, keepdims=True)


def _edge_pass1_body(g_ref, ea_ref, wc_ref, stats_ref):
    hb = g_ref[...].astype(F32) + jnp.dot(ea_ref[...], wc_ref[...],
                                          preferred_element_type=F32)
    _stats_accumulate(hb, stats_ref)


def _edge_pass1(g, ea, wc):
    e, l = g.shape
    br = _block_rows(e)
    return pl.pallas_call(
        _edge_pass1_body,
        grid=(e // br,),
        in_specs=[
            pl.BlockSpec((br, l), lambda i: (i, 0)),
            pl.BlockSpec((br, l), lambda i: (i, 0)),
            pl.BlockSpec((l, l), lambda i: (0, 0)),
        ],
        out_specs=pl.BlockSpec((2, l), lambda i: (0, 0)),
        out_shape=jax.ShapeDtypeStruct((2, l), F32),
        compiler_params=pltpu.CompilerParams(
            dimension_semantics=("arbitrary",)),
    )(g, ea, wc)


def _edge_pass2_body(g_ref, ea_ref, wc_ref, scale_ref, shift_ref, w2_ref,
                     b2_ref, out_ref):
    ea = ea_ref[...]
    hb = g_ref[...].astype(F32) + jnp.dot(ea, wc_ref[...],
                                          preferred_element_type=F32)
    hn = jnp.maximum(hb * scale_ref[...] + shift_ref[...], 0.0)
    y = jnp.dot(hn, w2_ref[...], preferred_element_type=F32) + b2_ref[...]
    out_ref[...] = jnp.maximum(y, 0.0) + ea


def _edge_pass2(g, ea, wc, scale, shift, w2, b2):
    e, l = g.shape
    br = _block_rows(e)
    row = pl.BlockSpec((br, l), lambda i: (i, 0))
    vec = pl.BlockSpec((1, l), lambda i: (0, 0))
    w = pl.BlockSpec((l, l), lambda i: (0, 0))
    return pl.pallas_call(
        _edge_pass2_body,
        grid=(e // br,),
        in_specs=[row, row, w, vec, vec, w, vec],
        out_specs=row,
        out_shape=jax.ShapeDtypeStruct((e, l), F32),
    )(g, ea, wc, scale, shift, w2, b2.reshape(1, l))


def _node_pass1_body(x_ref, a0_ref, a1_ref, wa_ref, wb_ref, b_ref,
                     h_ref, stats_ref):
    h = (jnp.dot(x_ref[...], wa_ref[...], preferred_element_type=F32)
         + jnp.dot(a0_ref[...] + a1_ref[...], wb_ref[...],
                   preferred_element_type=F32)
         + b_ref[...])
    h_ref[...] = h
    _stats_accumulate(h, stats_ref)


def _node_pass1(x, a0, a1, wa, wb, b1):
    n, l = x.shape
    br = _block_rows(n)
    row = pl.BlockSpec((br, l), lambda i: (i, 0))
    w = pl.BlockSpec((l, l), lambda i: (0, 0))
    return pl.pallas_call(
        _node_pass1_body,
        grid=(n // br,),
        in_specs=[row, row, row, w, w, pl.BlockSpec((1, l), lambda i: (0, 0))],
        out_specs=[row, pl.BlockSpec((2, l), lambda i: (0, 0))],
        out_shape=[jax.ShapeDtypeStruct((n, l), F32),
                   jax.ShapeDtypeStruct((2, l), F32)],
        compiler_params=pltpu.CompilerParams(
            dimension_semantics=("arbitrary",)),
    )(x, a0, a1, wa, wb, b1.reshape(1, l))


def _node_pass2_body(h_ref, res_ref, scale_ref, shift_ref, w2_ref, b2_ref,
                     out_ref):
    hn = jnp.maximum(h_ref[...] * scale_ref[...] + shift_ref[...], 0.0)
    y = jnp.dot(hn, w2_ref[...], preferred_element_type=F32) + b2_ref[...]
    out_ref[...] = jnp.maximum(y, 0.0) + res_ref[...]


def _node_pass2(h, res, scale, shift, w2, b2):
    r, l = h.shape
    br = _block_rows(r)
    row = pl.BlockSpec((br, l), lambda i: (i, 0))
    vec = pl.BlockSpec((1, l), lambda i: (0, 0))
    return pl.pallas_call(
        _node_pass2_body,
        grid=(r // br,),
        in_specs=[row, row, vec, vec,
                  pl.BlockSpec((l, l), lambda i: (0, 0)), vec],
        out_specs=row,
        out_shape=jax.ShapeDtypeStruct((r, l), F32),
    )(h, res, scale, shift, w2, b2.reshape(1, l))


def _bn_affine(stats, count, g1, be1, b1=None):
    mean = stats[0] / count
    var = stats[1] / count - mean * mean
    if b1 is not None:
        mean = mean + b1
    inv = g1 * lax.rsqrt(var + BN_EPS)
    scale = inv.reshape(1, -1)
    if b1 is not None:
        shift = (be1 + (b1 - mean) * inv).reshape(1, -1)
    else:
        shift = (be1 - mean * inv).reshape(1, -1)
    return scale, shift


_CG = 80


def _make_gather_add(n, e, l):
    nper = e // NW
    lw = l // 2
    assert nper % _CG == 0 and nper % 16 == 0
    nchunks = nper // _CG
    npairs = nchunks // 2
    has_tail = nchunks % 2 == 1
    assert npairs >= 2
    assert _CG % 16 == 0
    mesh = plsc.VectorSubcoreMesh(core_axis_name="c", subcore_axis_name="s",
                                  num_cores=NC, num_subcores=NS)

    @functools.partial(
        pl.kernel,
        out_type=jax.ShapeDtypeStruct((e, l), F32),
        mesh=mesh,
        compiler_params=pltpu.CompilerParams(needs_layout_passes=False,
                                             use_tc_tiling_on_sc=False),
        scratch_types=[
            pltpu.VMEM((nchunks, _CG), jnp.int32),
            pltpu.VMEM((nchunks, _CG), jnp.int32),
            pltpu.VMEM((_CG, l), jnp.bfloat16),
            pltpu.VMEM((_CG, l), jnp.bfloat16),
            pltpu.VMEM((_CG, l), F32),
            pltpu.VMEM((_CG, l), jnp.bfloat16),
            pltpu.VMEM((_CG, l), jnp.bfloat16),
            pltpu.VMEM((_CG, l), F32),
        ] + [pltpu.SemaphoreType.DMA] * 4,
    )
    def gather_add(p_hbm, q_hbm, dsts_hbm, srcs_hbm, g_hbm,
                   idxd, idxs, prow0, qrow0, orow0, prow1, qrow1, orow1,
                   semg0, semg1, semw0, semw1):
        wid = lax.axis_index("s") * NC + lax.axis_index("c")
        base0 = wid * nper
        pltpu.sync_copy(dsts_hbm.at[wid], idxd)
        pltpu.sync_copy(srcs_hbm.at[wid], idxs)

        bufs = ((prow0, qrow0, orow0, semg0, semw0),
                (prow1, qrow1, orow1, semg1, semw1))

        def fire_gather(c, b):
            prow, qrow, _, semg, _ = bufs[b]
            pltpu.async_copy(p_hbm.at[idxd.at[c]], prow, semg)
            pltpu.async_copy(q_hbm.at[idxs.at[c]], qrow, semg)

        def wait_gather(b):
            prow, qrow, _, semg, _ = bufs[b]
            pltpu.make_async_copy(p_hbm.at[idxd.at[0]], prow, semg).wait()
            pltpu.make_async_copy(q_hbm.at[idxs.at[0]], qrow, semg).wait()

        def add_rows(b):
            prow, qrow, orow, _, _ = bufs[b]

            @plsc.parallel_loop(0, _CG, 2, unroll=2)
            def _(r):
                for rr in range(2):
                    for j in range(l // (2 * LANES)):
                        base = j * 2 * LANES
                        sl = pl.ds(base, 2 * LANES)
                        a, b2 = plsc.unpack(prow[r + rr, sl] + qrow[r + rr, sl],
                                            format=plsc.PackFormat.INTERLEAVED)
                        orow[r + rr, pl.ds(base, LANES)] = a
                        orow[r + rr, pl.ds(base + LANES, LANES)] = b2

        def fire_wb(c, b):
            _, _, orow, _, semw = bufs[b]
            base = pl.multiple_of(base0 + c * _CG, 16)
            pltpu.async_copy(orow, g_hbm.at[pl.ds(base, _CG)], semw)

        def wait_wb(b):
            _, _, orow, _, semw = bufs[b]
            pltpu.make_async_copy(
                orow, g_hbm.at[pl.ds(base0, _CG)], semw).wait()

        fire_gather(0, 0)
        fire_gather(1, 1)
        wait_gather(0)
        add_rows(0)
        fire_gather(2, 0)
        fire_wb(0, 0)
        wait_gather(1)
        add_rows(1)
        fire_gather(3, 1)
        fire_wb(1, 1)

        def body(i, carry):
            for b in range(2):
                c = 2 * i + b
                wait_gather(b)
                wait_wb(b)
                add_rows(b)

                @pl.when(c + 2 < nchunks)
                def _():
                    fire_gather(c + 2, b)

                fire_wb(c, b)
            return carry

        lax.fori_loop(1, npairs, body, 0)

        if has_tail:
            wait_gather(0)
            wait_wb(0)
            add_rows(0)
            fire_wb(nchunks - 1, 0)
        wait_wb(0)
        wait_wb(1)

    return gather_add


def _make_scatter_sum(n, e, l):
    nper = e // NW
    nchunks = nper // _CG
    npairs = nchunks // 2
    has_tail = nchunks % 2 == 1
    zr = 64
    npad = -(-n // (NS * 128)) * NS * 128
    rpt = npad // NS
    tail_start = (n // rpt) * rpt
    tail_len = n - tail_start
    assert n % 8 == 0
    mesh = plsc.VectorSubcoreMesh(core_axis_name="c", subcore_axis_name="s",
                                  num_cores=NC, num_subcores=NS)

    @functools.partial(
        pl.kernel,
        out_type=[jax.ShapeDtypeStruct((n, l), F32)] * NC,
        mesh=mesh,
        scratch_types=[
            pltpu.VMEM((nchunks, _CG), jnp.int32),
            pltpu.VMEM((_CG, l), F32),
            pltpu.VMEM((_CG, l), F32),
            pltpu.VMEM((zr, l), F32),
            pltpu.VMEM_SHARED((npad, l), F32),
        ] + [pltpu.SemaphoreType.DMA] * 4,
    )
    def scatter_sum(upd_hbm, dsts_hbm, o0, o1,
                    idx_v, rows0, rows1, zeros_v, acc,
                    seml0, seml1, semsc0, semsc1):
        cid = lax.axis_index("c")
        sid = lax.axis_index("s")

        def zrow(r, c2):
            for j in range(l // LANES):
                zeros_v[r, pl.ds(j * LANES, LANES)] = jnp.zeros((LANES,), F32)
            return c2

        lax.fori_loop(0, zr, zrow, 0)
        row0 = sid * rpt
        for k in range(rpt // zr):
            pltpu.sync_copy(zeros_v, acc.at[pl.ds(row0 + k * zr, zr)])
        plsc.subcore_barrier()

        wid = cid * NS + sid
        base0 = wid * nper
        pltpu.sync_copy(dsts_hbm.at[wid], idx_v)

        bufs = ((rows0, seml0, semsc0), (rows1, seml1, semsc1))

        def fire_load(c, b):
            rows, seml, _ = bufs[b]
            base = pl.multiple_of(base0 + c * _CG, 8)
            pltpu.async_copy(upd_hbm.at[pl.ds(base, _CG)], rows, seml)

        def wait_load(b):
            rows, seml, _ = bufs[b]
            pltpu.make_async_copy(
                upd_hbm.at[pl.ds(base0, _CG)], rows, seml).wait()

        def fire_scat(c, b):
            rows, _, semsc = bufs[b]
            pltpu.async_copy(rows, acc.at[idx_v.at[c]], semsc, add=True)

        def wait_scat(b):
            rows, _, semsc = bufs[b]
            pltpu.make_async_copy(rows, acc.at[idx_v.at[0]], semsc).wait()

        fire_load(0, 0)
        fire_load(1, 1)

        def body(i, carry):
            for b in range(2):
                wait_load(b)
                fire_scat(2 * i + b, b)
            for b in range(2):
                c = 2 * i + b
                wait_scat(b)

                @pl.when(c + 2 < nchunks)
                def _():
                    fire_load(c + 2, b)
            return carry

        lax.fori_loop(0, npairs, body, 0)

        if has_tail:
            wait_load(0)
            fire_scat(nchunks - 1, 0)
            wait_scat(0)
        plsc.subcore_barrier()

        def copy_out(out_ref):
            @pl.when(row0 + rpt <= n)
            def _():
                pltpu.sync_copy(acc.at[pl.ds(row0, rpt)],
                                out_ref.at[pl.ds(row0, rpt)])
            if tail_len > 0:
                @pl.when(row0 == tail_start)
                def _():
                    pltpu.sync_copy(acc.at[pl.ds(tail_start, tail_len)],
                                    out_ref.at[pl.ds(tail_start, tail_len)])

        @pl.when(cid == 0)
        def _():
            copy_out(o0)

        @pl.when(cid == 1)
        def _():
            copy_out(o1)

    return scatter_sum


def _pack_bf16_pairs(t):
    n, l = t.shape
    tb = t.astype(jnp.bfloat16).reshape(n, l // 2, 2)
    return jax.lax.bitcast_convert_type(tb, jnp.int32)


def _unpack_perm(l):
    perm = []
    for j in range(l // (2 * LANES)):
        base = j * 2 * LANES
        perm += [base + 2 * i for i in range(LANES)]
        perm += [base + 2 * i + 1 for i in range(LANES)]
    return jnp.asarray(perm, jnp.int32)


def kernel(x, edge_attr, params, edge_index):
    n, l = x.shape
    e = edge_attr.shape[0]
    nper = e // NW
    src = edge_index[0].reshape(NW, nper // _CG, _CG)
    dst = edge_index[1].reshape(NW, nper // _CG, _CG)

    gather_add = _make_gather_add(n, e, l)
    scatter_sum = _make_scatter_sum(n, e, l)

    perm = _unpack_perm(l)
    for layer in params:
        ep, npar = layer['edge'], layer['node']
        wa, wb = ep['W1'][:l], ep['W1'][l:2 * l]
        wc = ep['W1'][2 * l:][:, perm]
        p, q = _compute_pq(x, wa, wb)
        g = gather_add(p, q, dst, src)
        stats = _edge_pass1(g, edge_attr, wc)
        scale, shift = _bn_affine(stats, e, ep['g1'][perm], ep['be1'][perm],
                                  ep['b1'][perm])
        upd = _edge_pass2(g, edge_attr, wc, scale, shift, ep['W2'][perm, :],
                          ep['b2'])

        a0, a1 = scatter_sum(upd, dst)

        hn, nstats = _node_pass1(x, a0, a1, npar['W1'][:l], npar['W1'][l:],
                                 npar['b1'])
        nscale, nshift = _bn_affine(nstats, n, npar['g1'], npar['be1'])
        x = _node_pass2(hn, x, nscale, nshift, npar['W2'], npar['b2'])
        edge_attr = upd

    return x, edge_attr


# speedup vs baseline: 1.0722x; 1.0084x over previous
import functools

import jax
import jax.numpy as jnp
from jax import lax
from jax.experimental import pallas as pl
from jax.experimental.pallas import tpu as pltpu
from jax.experimental.pallas import tpu_sc as plsc

F32 = jnp.float32
BN_EPS = 1e-5
NC, NS, LANES = 2, 16, 16
NW = NC * NS


def _block_rows(rows, cap=2048):
    best = 8
    for b in range(8, cap + 1, 8):
        if rows % b == 0:
            best = b
    return best


def _pq_body(x_ref, wa_ref, wb_ref, p_ref, q_ref):
    x = x_ref[...]
    p_ref[...] = jnp.dot(x, wa_ref[...],
                         preferred_element_type=F32).astype(jnp.bfloat16)
    q_ref[...] = jnp.dot(x, wb_ref[...],
                         preferred_element_type=F32).astype(jnp.bfloat16)


def _compute_pq(x, wa, wb):
    n, l = x.shape
    br = _block_rows(n)
    return pl.pallas_call(
        _pq_body,
        grid=(n // br,),
        in_specs=[
            pl.BlockSpec((br, l), lambda i: (i, 0)),
            pl.BlockSpec((l, l), lambda i: (0, 0)),
            pl.BlockSpec((l, l), lambda i: (0, 0)),
        ],
        out_specs=[
            pl.BlockSpec((br, l), lambda i: (i, 0)),
            pl.BlockSpec((br, l), lambda i: (i, 0)),
        ],
        out_shape=[jax.ShapeDtypeStruct((n, l), jnp.bfloat16)] * 2,
    )(x, wa, wb)


def _stats_accumulate(h, stats_ref):
    @pl.when(pl.program_id(0) == 0)
    def _():
        stats_ref[...] = jnp.zeros_like(stats_ref)

    stats_ref[0:1, :] += jnp.sum(h, axis=0, keepdims=True)
    stats_ref[1:2, :] += jnp.sum(h * h, axis=0, keepdims=True)


def _edge_pass1_body(g_ref, ea_ref, wc_ref, stats_ref):
    hb = g_ref[...].astype(F32) + jnp.dot(ea_ref[...], wc_ref[...],
                                          preferred_element_type=F32)
    _stats_accumulate(hb, stats_ref)


def _edge_pass1(g, ea, wc):
    e, l = g.shape
    br = _block_rows(e)
    return pl.pallas_call(
        _edge_pass1_body,
        grid=(e // br,),
        in_specs=[
            pl.BlockSpec((br, l), lambda i: (i, 0)),
            pl.BlockSpec((br, l), lambda i: (i, 0)),
            pl.BlockSpec((l, l), lambda i: (0, 0)),
        ],
        out_specs=pl.BlockSpec((2, l), lambda i: (0, 0)),
        out_shape=jax.ShapeDtypeStruct((2, l), F32),
        compiler_params=pltpu.CompilerParams(
            dimension_semantics=("arbitrary",)),
    )(g, ea, wc)


def _bn_scale_shift(stats_ref, count, g1_ref, be1_ref):
    mean = stats_ref[0:1, :] / count
    var = stats_ref[1:2, :] / count - mean * mean
    scale = g1_ref[...] * lax.rsqrt(var + BN_EPS)
    shift = be1_ref[...] - mean * scale
    return scale, shift


def _edge_pass2_body(g_ref, ea_ref, wc_ref, stats_ref, g1_ref, be1_ref,
                     w2_ref, b2_ref, out_ref, *, count):
    scale, shift = _bn_scale_shift(stats_ref, count, g1_ref, be1_ref)
    ea = ea_ref[...]
    hb = g_ref[...].astype(F32) + jnp.dot(ea, wc_ref[...],
                                          preferred_element_type=F32)
    hn = jnp.maximum(hb * scale + shift, 0.0)
    y = jnp.dot(hn, w2_ref[...], preferred_element_type=F32) + b2_ref[...]
    out_ref[...] = jnp.maximum(y, 0.0) + ea


def _edge_pass2(g, ea, wc, stats, g1, be1, w2, b2):
    e, l = g.shape
    br = _block_rows(e)
    row = pl.BlockSpec((br, l), lambda i: (i, 0))
    vec = pl.BlockSpec((1, l), lambda i: (0, 0))
    w = pl.BlockSpec((l, l), lambda i: (0, 0))
    return pl.pallas_call(
        functools.partial(_edge_pass2_body, count=float(e)),
        grid=(e // br,),
        in_specs=[row, row, w, pl.BlockSpec((2, l), lambda i: (0, 0)),
                  vec, vec, w, vec],
        out_specs=row,
        out_shape=jax.ShapeDtypeStruct((e, l), F32),
    )(g, ea, wc, stats, g1.reshape(1, l), be1.reshape(1, l),
      w2, b2.reshape(1, l))


def _node_pass1_body(x_ref, a0_ref, a1_ref, wa_ref, wb_ref,
                     h_ref, stats_ref):
    h = (jnp.dot(x_ref[...], wa_ref[...], preferred_element_type=F32)
         + jnp.dot(a0_ref[...] + a1_ref[...], wb_ref[...],
                   preferred_element_type=F32))
    h_ref[...] = h
    _stats_accumulate(h, stats_ref)


def _node_pass1(x, a0, a1, wa, wb):
    n, l = x.shape
    br = _block_rows(n)
    row = pl.BlockSpec((br, l), lambda i: (i, 0))
    w = pl.BlockSpec((l, l), lambda i: (0, 0))
    return pl.pallas_call(
        _node_pass1_body,
        grid=(n // br,),
        in_specs=[row, row, row, w, w],
        out_specs=[row, pl.BlockSpec((2, l), lambda i: (0, 0))],
        out_shape=[jax.ShapeDtypeStruct((n, l), F32),
                   jax.ShapeDtypeStruct((2, l), F32)],
        compiler_params=pltpu.CompilerParams(
            dimension_semantics=("arbitrary",)),
    )(x, a0, a1, wa, wb)


def _node_pass2_out(h_ref, res_ref, stats_ref, g1_ref, be1_ref, w2_ref,
                    b2_ref, count):
    scale, shift = _bn_scale_shift(stats_ref, count, g1_ref, be1_ref)
    hn = jnp.maximum(h_ref[...] * scale + shift, 0.0)
    y = jnp.dot(hn, w2_ref[...], preferred_element_type=F32) + b2_ref[...]
    return jnp.maximum(y, 0.0) + res_ref[...]


def _node_pass2_body(h_ref, res_ref, stats_ref, g1_ref, be1_ref, w2_ref,
                     b2_ref, out_ref, *, count):
    out_ref[...] = _node_pass2_out(h_ref, res_ref, stats_ref, g1_ref,
                                   be1_ref, w2_ref, b2_ref, count)


def _node_pass2_pq_body(h_ref, res_ref, stats_ref, g1_ref, be1_ref, w2_ref,
                        b2_ref, wan_ref, wbn_ref, out_ref, p_ref, q_ref, *,
                        count):
    xn = _node_pass2_out(h_ref, res_ref, stats_ref, g1_ref, be1_ref, w2_ref,
                         b2_ref, count)
    out_ref[...] = xn
    p_ref[...] = jnp.dot(xn, wan_ref[...],
                         preferred_element_type=F32).astype(jnp.bfloat16)
    q_ref[...] = jnp.dot(xn, wbn_ref[...],
                         preferred_element_type=F32).astype(jnp.bfloat16)


def _node_pass2(h, res, stats, g1, be1, w2, b2, wan=None, wbn=None):
    r, l = h.shape
    br = _block_rows(r)
    row = pl.BlockSpec((br, l), lambda i: (i, 0))
    vec = pl.BlockSpec((1, l), lambda i: (0, 0))
    w = pl.BlockSpec((l, l), lambda i: (0, 0))
    stat = pl.BlockSpec((2, l), lambda i: (0, 0))
    args = (h, res, stats, g1.reshape(1, l), be1.reshape(1, l), w2,
            b2.reshape(1, l))
    if wan is None:
        return pl.pallas_call(
            functools.partial(_node_pass2_body, count=float(r)),
            grid=(r // br,),
            in_specs=[row, row, stat, vec, vec, w, vec],
            out_specs=row,
            out_shape=jax.ShapeDtypeStruct((r, l), F32),
        )(*args)
    return pl.pallas_call(
        functools.partial(_node_pass2_pq_body, count=float(r)),
        grid=(r // br,),
        in_specs=[row, row, stat, vec, vec, w, vec, w, w],
        out_specs=[row, row, row],
        out_shape=[jax.ShapeDtypeStruct((r, l), F32),
                   jax.ShapeDtypeStruct((r, l), jnp.bfloat16),
                   jax.ShapeDtypeStruct((r, l), jnp.bfloat16)],
    )(*args, wan, wbn)


_CG = 80


def _make_gather_add(n, e, l):
    nper = e // NW
    lw = l // 2
    assert nper % _CG == 0 and nper % 16 == 0
    nchunks = nper // _CG
    npairs = nchunks // 2
    has_tail = nchunks % 2 == 1
    assert npairs >= 2
    assert _CG % 16 == 0
    mesh = plsc.VectorSubcoreMesh(core_axis_name="c", subcore_axis_name="s",
                                  num_cores=NC, num_subcores=NS)

    @functools.partial(
        pl.kernel,
        out_type=jax.ShapeDtypeStruct((e, l), F32),
        mesh=mesh,
        compiler_params=pltpu.CompilerParams(needs_layout_passes=False,
                                             use_tc_tiling_on_sc=False),
        scratch_types=[
            pltpu.VMEM((nchunks, _CG), jnp.int32),
            pltpu.VMEM((nchunks, _CG), jnp.int32),
            pltpu.VMEM((_CG, l), jnp.bfloat16),
            pltpu.VMEM((_CG, l), jnp.bfloat16),
            pltpu.VMEM((_CG, l), F32),
            pltpu.VMEM((_CG, l), jnp.bfloat16),
            pltpu.VMEM((_CG, l), jnp.bfloat16),
            pltpu.VMEM((_CG, l), F32),
        ] + [pltpu.SemaphoreType.DMA] * 4,
    )
    def gather_add(p_hbm, q_hbm, dsts_hbm, srcs_hbm, g_hbm,
                   idxd, idxs, prow0, qrow0, orow0, prow1, qrow1, orow1,
                   semg0, semg1, semw0, semw1):
        wid = lax.axis_index("s") * NC + lax.axis_index("c")
        base0 = wid * nper
        pltpu.sync_copy(dsts_hbm.at[wid], idxd)
        pltpu.sync_copy(srcs_hbm.at[wid], idxs)

        bufs = ((prow0, qrow0, orow0, semg0, semw0),
                (prow1, qrow1, orow1, semg1, semw1))

        def fire_gather(c, b):
            prow, qrow, _, semg, _ = bufs[b]
            pltpu.async_copy(p_hbm.at[idxd.at[c]], prow, semg)
            pltpu.async_copy(q_hbm.at[idxs.at[c]], qrow, semg)

        def wait_gather(b):
            prow, qrow, _, semg, _ = bufs[b]
            pltpu.make_async_copy(p_hbm.at[idxd.at[0]], prow, semg).wait()
            pltpu.make_async_copy(q_hbm.at[idxs.at[0]], qrow, semg).wait()

        def add_rows(b):
            prow, qrow, orow, _, _ = bufs[b]

            @plsc.parallel_loop(0, _CG, 2, unroll=2)
            def _(r):
                for rr in range(2):
                    for j in range(l // (2 * LANES)):
                        base = j * 2 * LANES
                        sl = pl.ds(base, 2 * LANES)
                        a, b2 = plsc.unpack(prow[r + rr, sl] + qrow[r + rr, sl],
                                            format=plsc.PackFormat.INTERLEAVED)
                        orow[r + rr, pl.ds(base, LANES)] = a
                        orow[r + rr, pl.ds(base + LANES, LANES)] = b2

        def fire_wb(c, b):
            _, _, orow, _, semw = bufs[b]
            base = pl.multiple_of(base0 + c * _CG, 16)
            pltpu.async_copy(orow, g_hbm.at[pl.ds(base, _CG)], semw)

        def wait_wb(b):
            _, _, orow, _, semw = bufs[b]
            pltpu.make_async_copy(
                orow, g_hbm.at[pl.ds(base0, _CG)], semw).wait()

        fire_gather(0, 0)
        fire_gather(1, 1)
        wait_gather(0)
        add_rows(0)
        fire_gather(2, 0)
        fire_wb(0, 0)
        wait_gather(1)
        add_rows(1)
        fire_gather(3, 1)
        fire_wb(1, 1)

        def body(i, carry):
            for b in range(2):
                c = 2 * i + b
                wait_gather(b)
                wait_wb(b)
                add_rows(b)

                @pl.when(c + 2 < nchunks)
                def _():
                    fire_gather(c + 2, b)

                fire_wb(c, b)
            return carry

        lax.fori_loop(1, npairs, body, 0)

        if has_tail:
            wait_gather(0)
            wait_wb(0)
            add_rows(0)
            fire_wb(nchunks - 1, 0)
        wait_wb(0)
        wait_wb(1)

    return gather_add


def _make_scatter_sum(n, e, l):
    nper = e // NW
    nchunks = nper // _CG
    npairs = nchunks // 2
    has_tail = nchunks % 2 == 1
    zr = 64
    npad = -(-n // (NS * 128)) * NS * 128
    rpt = npad // NS
    tail_start = (n // rpt) * rpt
    tail_len = n - tail_start
    assert n % 8 == 0
    mesh = plsc.VectorSubcoreMesh(core_axis_name="c", subcore_axis_name="s",
                                  num_cores=NC, num_subcores=NS)

    @functools.partial(
        pl.kernel,
        out_type=[jax.ShapeDtypeStruct((n, l), F32)] * NC,
        mesh=mesh,
        scratch_types=[
            pltpu.VMEM((nchunks, _CG), jnp.int32),
            pltpu.VMEM((_CG, l), F32),
            pltpu.VMEM((_CG, l), F32),
            pltpu.VMEM((zr, l), F32),
            pltpu.VMEM_SHARED((npad, l), F32),
        ] + [pltpu.SemaphoreType.DMA] * 4,
    )
    def scatter_sum(upd_hbm, dsts_hbm, o0, o1,
                    idx_v, rows0, rows1, zeros_v, acc,
                    seml0, seml1, semsc0, semsc1):
        cid = lax.axis_index("c")
        sid = lax.axis_index("s")

        def zrow(r, c2):
            for j in range(l // LANES):
                zeros_v[r, pl.ds(j * LANES, LANES)] = jnp.zeros((LANES,), F32)
            return c2

        lax.fori_loop(0, zr, zrow, 0)
        row0 = sid * rpt
        for k in range(rpt // zr):
            pltpu.sync_copy(zeros_v, acc.at[pl.ds(row0 + k * zr, zr)])
        plsc.subcore_barrier()

        wid = cid * NS + sid
        base0 = wid * nper
        pltpu.sync_copy(dsts_hbm.at[wid], idx_v)

        bufs = ((rows0, seml0, semsc0), (rows1, seml1, semsc1))

        def fire_load(c, b):
            rows, seml, _ = bufs[b]
            base = pl.multiple_of(base0 + c * _CG, 8)
            pltpu.async_copy(upd_hbm.at[pl.ds(base, _CG)], rows, seml)

        def wait_load(b):
            rows, seml, _ = bufs[b]
            pltpu.make_async_copy(
                upd_hbm.at[pl.ds(base0, _CG)], rows, seml).wait()

        def fire_scat(c, b):
            rows, _, semsc = bufs[b]
            pltpu.async_copy(rows, acc.at[idx_v.at[c]], semsc, add=True)

        def wait_scat(b):
            rows, _, semsc = bufs[b]
            pltpu.make_async_copy(rows, acc.at[idx_v.at[0]], semsc).wait()

        fire_load(0, 0)
        fire_load(1, 1)

        def body(i, carry):
            for b in range(2):
                wait_load(b)
                fire_scat(2 * i + b, b)
            for b in range(2):
                c = 2 * i + b
                wait_scat(b)

                @pl.when(c + 2 < nchunks)
                def _():
                    fire_load(c + 2, b)
            return carry

        lax.fori_loop(0, npairs, body, 0)

        if has_tail:
            wait_load(0)
            fire_scat(nchunks - 1, 0)
            wait_scat(0)
        plsc.subcore_barrier()

        def copy_out(out_ref):
            @pl.when(row0 + rpt <= n)
            def _():
                pltpu.sync_copy(acc.at[pl.ds(row0, rpt)],
                                out_ref.at[pl.ds(row0, rpt)])
            if tail_len > 0:
                @pl.when(row0 == tail_start)
                def _():
                    pltpu.sync_copy(acc.at[pl.ds(tail_start, tail_len)],
                                    out_ref.at[pl.ds(tail_start, tail_len)])

        @pl.when(cid == 0)
        def _():
            copy_out(o0)

        @pl.when(cid == 1)
        def _():
            copy_out(o1)

    return scatter_sum


def _unpack_perm(l):
    perm = []
    for j in range(l // (2 * LANES)):
        base = j * 2 * LANES
        perm += [base + 2 * i for i in range(LANES)]
        perm += [base + 2 * i + 1 for i in range(LANES)]
    return jnp.asarray(perm, jnp.int32)


def kernel(x, edge_attr, params, edge_index):
    n, l = x.shape
    e = edge_attr.shape[0]
    nper = e // NW
    src = edge_index[0].reshape(NW, nper // _CG, _CG)
    dst = edge_index[1].reshape(NW, nper // _CG, _CG)

    gather_add = _make_gather_add(n, e, l)
    scatter_sum = _make_scatter_sum(n, e, l)

    perm = _unpack_perm(l)
    p = q = None
    for li, layer in enumerate(params):
        ep, npar = layer['edge'], layer['node']
        wc = ep['W1'][2 * l:][:, perm]
        if p is None:
            p, q = _compute_pq(x, ep['W1'][:l], ep['W1'][l:2 * l])
        g = gather_add(p, q, dst, src)
        stats = _edge_pass1(g, edge_attr, wc)
        upd = _edge_pass2(g, edge_attr, wc, stats, ep['g1'][perm],
                          ep['be1'][perm], ep['W2'][perm, :], ep['b2'])

        a0, a1 = scatter_sum(upd, dst)

        hn, nstats = _node_pass1(x, a0, a1, npar['W1'][:l], npar['W1'][l:])
        if li + 1 < len(params):
            nep = params[li + 1]['edge']
            x, p, q = _node_pass2(hn, x, nstats, npar['g1'], npar['be1'],
                                  npar['W2'], npar['b2'],
                                  nep['W1'][:l], nep['W1'][l:2 * l])
        else:
            x = _node_pass2(hn, x, nstats, npar['g1'], npar['be1'],
                            npar['W2'], npar['b2'])
        edge_attr = upd

    return x, edge_attr


# speedup vs baseline: 1.2453x; 1.1615x over previous
import functools

import jax
import jax.numpy as jnp
from jax import lax
from jax.experimental import pallas as pl
from jax.experimental.pallas import tpu as pltpu
from jax.experimental.pallas import tpu_sc as plsc

F32 = jnp.float32
BN_EPS = 1e-5
NC, NS, LANES = 2, 16, 16
NW = NC * NS


def _block_rows(rows, cap=4096):
    best = 8
    for b in range(8, cap + 1, 8):
        if rows % b == 0:
            best = b
    return best


def _pq_body(x_ref, wa_ref, wb_ref, p_ref, q_ref):
    x = x_ref[...]
    p_ref[...] = jnp.dot(x, wa_ref[...],
                         preferred_element_type=F32).astype(jnp.bfloat16)
    q_ref[...] = jnp.dot(x, wb_ref[...],
                         preferred_element_type=F32).astype(jnp.bfloat16)


def _compute_pq(x, wa, wb):
    n, l = x.shape
    br = _block_rows(n)
    return pl.pallas_call(
        _pq_body,
        grid=(n // br,),
        in_specs=[
            pl.BlockSpec((br, l), lambda i: (i, 0)),
            pl.BlockSpec((l, l), lambda i: (0, 0)),
            pl.BlockSpec((l, l), lambda i: (0, 0)),
        ],
        out_specs=[
            pl.BlockSpec((br, l), lambda i: (i, 0)),
            pl.BlockSpec((br, l), lambda i: (i, 0)),
        ],
        out_shape=[jax.ShapeDtypeStruct((n, l), jnp.bfloat16)] * 2,
    )(x, wa, wb)


def _stats_accumulate(h, stats_ref):
    @pl.when(pl.program_id(0) == 0)
    def _():
        stats_ref[...] = jnp.zeros_like(stats_ref)

    stats_ref[0:1, :] += jnp.sum(h, axis=0, keepdims=True)
    stats_ref[1:2, :] += jnp.sum(h * h, axis=0, keepdims=True)


def _edge_pass1_body(g_ref, ea_ref, wc_ref, stats_ref):
    hb = g_ref[...].astype(F32) + jnp.dot(ea_ref[...], wc_ref[...],
                                          preferred_element_type=F32)
    _stats_accumulate(hb, stats_ref)


def _edge_pass1(g, ea, wc):
    e, l = g.shape
    br = _block_rows(e)
    return pl.pallas_call(
        _edge_pass1_body,
        grid=(e // br,),
        in_specs=[
            pl.BlockSpec((br, l), lambda i: (i, 0)),
            pl.BlockSpec((br, l), lambda i: (i, 0)),
            pl.BlockSpec((l, l), lambda i: (0, 0)),
        ],
        out_specs=pl.BlockSpec((2, l), lambda i: (0, 0)),
        out_shape=jax.ShapeDtypeStruct((2, l), F32),
        compiler_params=pltpu.CompilerParams(
            dimension_semantics=("arbitrary",)),
    )(g, ea, wc)


def _bn_scale_shift(stats_ref, count, g1_ref, be1_ref):
    mean = stats_ref[0:1, :] / count
    var = stats_ref[1:2, :] / count - mean * mean
    scale = g1_ref[...] * lax.rsqrt(var + BN_EPS)
    shift = be1_ref[...] - mean * scale
    return scale, shift


def _edge_pass2_body(g_ref, ea_ref, wc_ref, stats_ref, g1_ref, be1_ref,
                     w2_ref, b2_ref, out_ref, *, count):
    scale, shift = _bn_scale_shift(stats_ref, count, g1_ref, be1_ref)
    ea = ea_ref[...]
    hb = g_ref[...].astype(F32) + jnp.dot(ea, wc_ref[...],
                                          preferred_element_type=F32)
    hn = jnp.maximum(hb * scale + shift, 0.0)
    y = jnp.dot(hn, w2_ref[...], preferred_element_type=F32) + b2_ref[...]
    out_ref[...] = jnp.maximum(y, 0.0) + ea


def _edge_pass2(g, ea, wc, stats, g1, be1, w2, b2):
    e, l = g.shape
    br = _block_rows(e)
    row = pl.BlockSpec((br, l), lambda i: (i, 0))
    vec = pl.BlockSpec((1, l), lambda i: (0, 0))
    w = pl.BlockSpec((l, l), lambda i: (0, 0))
    return pl.pallas_call(
        functools.partial(_edge_pass2_body, count=float(e)),
        grid=(e // br,),
        in_specs=[row, row, w, pl.BlockSpec((2, l), lambda i: (0, 0)),
                  vec, vec, w, vec],
        out_specs=row,
        out_shape=jax.ShapeDtypeStruct((e, l), F32),
    )(g, ea, wc, stats, g1.reshape(1, l), be1.reshape(1, l),
      w2, b2.reshape(1, l))


def _node_pass1_body(x_ref, a0_ref, a1_ref, wa_ref, wb_ref,
                     h_ref, stats_ref):
    h = (jnp.dot(x_ref[...], wa_ref[...], preferred_element_type=F32)
         + jnp.dot(a0_ref[...] + a1_ref[...], wb_ref[...],
                   preferred_element_type=F32))
    h_ref[...] = h
    _stats_accumulate(h, stats_ref)


def _node_pass1(x, a0, a1, wa, wb):
    n, l = x.shape
    br = _block_rows(n)
    row = pl.BlockSpec((br, l), lambda i: (i, 0))
    w = pl.BlockSpec((l, l), lambda i: (0, 0))
    return pl.pallas_call(
        _node_pass1_body,
        grid=(n // br,),
        in_specs=[row, row, row, w, w],
        out_specs=[row, pl.BlockSpec((2, l), lambda i: (0, 0))],
        out_shape=[jax.ShapeDtypeStruct((n, l), F32),
                   jax.ShapeDtypeStruct((2, l), F32)],
        compiler_params=pltpu.CompilerParams(
            dimension_semantics=("arbitrary",)),
    )(x, a0, a1, wa, wb)


def _node_pass2_out(h_ref, res_ref, stats_ref, g1_ref, be1_ref, w2_ref,
                    b2_ref, count):
    scale, shift = _bn_scale_shift(stats_ref, count, g1_ref, be1_ref)
    hn = jnp.maximum(h_ref[...] * scale + shift, 0.0)
    y = jnp.dot(hn, w2_ref[...], preferred_element_type=F32) + b2_ref[...]
    return jnp.maximum(y, 0.0) + res_ref[...]


def _node_pass2_body(h_ref, res_ref, stats_ref, g1_ref, be1_ref, w2_ref,
                     b2_ref, out_ref, *, count):
    out_ref[...] = _node_pass2_out(h_ref, res_ref, stats_ref, g1_ref,
                                   be1_ref, w2_ref, b2_ref, count)


def _node_pass2_pq_body(h_ref, res_ref, stats_ref, g1_ref, be1_ref, w2_ref,
                        b2_ref, wan_ref, wbn_ref, out_ref, p_ref, q_ref, *,
                        count):
    xn = _node_pass2_out(h_ref, res_ref, stats_ref, g1_ref, be1_ref, w2_ref,
                         b2_ref, count)
    out_ref[...] = xn
    p_ref[...] = jnp.dot(xn, wan_ref[...],
                         preferred_element_type=F32).astype(jnp.bfloat16)
    q_ref[...] = jnp.dot(xn, wbn_ref[...],
                         preferred_element_type=F32).astype(jnp.bfloat16)


def _node_pass2(h, res, stats, g1, be1, w2, b2, wan=None, wbn=None):
    r, l = h.shape
    br = _block_rows(r)
    row = pl.BlockSpec((br, l), lambda i: (i, 0))
    vec = pl.BlockSpec((1, l), lambda i: (0, 0))
    w = pl.BlockSpec((l, l), lambda i: (0, 0))
    stat = pl.BlockSpec((2, l), lambda i: (0, 0))
    args = (h, res, stats, g1.reshape(1, l), be1.reshape(1, l), w2,
            b2.reshape(1, l))
    if wan is None:
        return pl.pallas_call(
            functools.partial(_node_pass2_body, count=float(r)),
            grid=(r // br,),
            in_specs=[row, row, stat, vec, vec, w, vec],
            out_specs=row,
            out_shape=jax.ShapeDtypeStruct((r, l), F32),
        )(*args)
    return pl.pallas_call(
        functools.partial(_node_pass2_pq_body, count=float(r)),
        grid=(r // br,),
        in_specs=[row, row, stat, vec, vec, w, vec, w, w],
        out_specs=[row, row, row],
        out_shape=[jax.ShapeDtypeStruct((r, l), F32),
                   jax.ShapeDtypeStruct((r, l), jnp.bfloat16),
                   jax.ShapeDtypeStruct((r, l), jnp.bfloat16)],
    )(*args, wan, wbn)


_CG = 80


def _make_gather_add(n, e, l):
    nper = e // NW
    lw = l // 2
    assert nper % _CG == 0 and nper % 16 == 0
    nchunks = nper // _CG
    npairs = nchunks // 2
    has_tail = nchunks % 2 == 1
    assert npairs >= 2
    assert _CG % 16 == 0
    mesh = plsc.VectorSubcoreMesh(core_axis_name="c", subcore_axis_name="s",
                                  num_cores=NC, num_subcores=NS)

    @functools.partial(
        pl.kernel,
        out_type=jax.ShapeDtypeStruct((e, l), F32),
        mesh=mesh,
        compiler_params=pltpu.CompilerParams(needs_layout_passes=False,
                                             use_tc_tiling_on_sc=False),
        scratch_types=[
            pltpu.VMEM((nchunks, _CG), jnp.int32),
            pltpu.VMEM((nchunks, _CG), jnp.int32),
            pltpu.VMEM((_CG, l), jnp.bfloat16),
            pltpu.VMEM((_CG, l), jnp.bfloat16),
            pltpu.VMEM((_CG, l), F32),
            pltpu.VMEM((_CG, l), jnp.bfloat16),
            pltpu.VMEM((_CG, l), jnp.bfloat16),
            pltpu.VMEM((_CG, l), F32),
        ] + [pltpu.SemaphoreType.DMA] * 4,
    )
    def gather_add(p_hbm, q_hbm, dsts_hbm, srcs_hbm, g_hbm,
                   idxd, idxs, prow0, qrow0, orow0, prow1, qrow1, orow1,
                   semg0, semg1, semw0, semw1):
        wid = lax.axis_index("s") * NC + lax.axis_index("c")
        base0 = wid * nper
        pltpu.sync_copy(dsts_hbm.at[wid], idxd)
        pltpu.sync_copy(srcs_hbm.at[wid], idxs)

        bufs = ((prow0, qrow0, orow0, semg0, semw0),
                (prow1, qrow1, orow1, semg1, semw1))

        def fire_gather(c, b):
            prow, qrow, _, semg, _ = bufs[b]
            pltpu.async_copy(p_hbm.at[idxd.at[c]], prow, semg)
            pltpu.async_copy(q_hbm.at[idxs.at[c]], qrow, semg)

        def wait_gather(b):
            prow, qrow, _, semg, _ = bufs[b]
            pltpu.make_async_copy(p_hbm.at[idxd.at[0]], prow, semg).wait()
            pltpu.make_async_copy(q_hbm.at[idxs.at[0]], qrow, semg).wait()

        def add_rows(b):
            prow, qrow, orow, _, _ = bufs[b]

            @plsc.parallel_loop(0, _CG, 2, unroll=2)
            def _(r):
                for rr in range(2):
                    for j in range(l // (2 * LANES)):
                        base = j * 2 * LANES
                        sl = pl.ds(base, 2 * LANES)
                        a, b2 = plsc.unpack(prow[r + rr, sl] + qrow[r + rr, sl],
                                            format=plsc.PackFormat.INTERLEAVED)
                        orow[r + rr, pl.ds(base, LANES)] = a
                        orow[r + rr, pl.ds(base + LANES, LANES)] = b2

        def fire_wb(c, b):
            _, _, orow, _, semw = bufs[b]
            base = pl.multiple_of(base0 + c * _CG, 16)
            pltpu.async_copy(orow, g_hbm.at[pl.ds(base, _CG)], semw)

        def wait_wb(b):
            _, _, orow, _, semw = bufs[b]
            pltpu.make_async_copy(
                orow, g_hbm.at[pl.ds(base0, _CG)], semw).wait()

        fire_gather(0, 0)
        fire_gather(1, 1)
        wait_gather(0)
        add_rows(0)
        fire_gather(2, 0)
        fire_wb(0, 0)
        wait_gather(1)
        add_rows(1)
        fire_gather(3, 1)
        fire_wb(1, 1)

        def body(i, carry):
            for b in range(2):
                c = 2 * i + b
                wait_gather(b)
                wait_wb(b)
                add_rows(b)

                @pl.when(c + 2 < nchunks)
                def _():
                    fire_gather(c + 2, b)

                fire_wb(c, b)
            return carry

        lax.fori_loop(1, npairs, body, 0)

        if has_tail:
            wait_gather(0)
            wait_wb(0)
            add_rows(0)
            fire_wb(nchunks - 1, 0)
        wait_wb(0)
        wait_wb(1)

    return gather_add


def _make_scatter_sum(n, e, l):
    nper = e // NW
    nchunks = nper // _CG
    npairs = nchunks // 2
    has_tail = nchunks % 2 == 1
    zr = 64
    npad = -(-n // (NS * 128)) * NS * 128
    rpt = npad // NS
    tail_start = (n // rpt) * rpt
    tail_len = n - tail_start
    assert n % 8 == 0
    mesh = plsc.VectorSubcoreMesh(core_axis_name="c", subcore_axis_name="s",
                                  num_cores=NC, num_subcores=NS)

    @functools.partial(
        pl.kernel,
        out_type=[jax.ShapeDtypeStruct((n, l), F32)] * NC,
        mesh=mesh,
        scratch_types=[
            pltpu.VMEM((nchunks, _CG), jnp.int32),
            pltpu.VMEM((_CG, l), F32),
            pltpu.VMEM((_CG, l), F32),
            pltpu.VMEM((zr, l), F32),
            pltpu.VMEM_SHARED((npad, l), F32),
        ] + [pltpu.SemaphoreType.DMA] * 4,
    )
    def scatter_sum(upd_hbm, dsts_hbm, o0, o1,
                    idx_v, rows0, rows1, zeros_v, acc,
                    seml0, seml1, semsc0, semsc1):
        cid = lax.axis_index("c")
        sid = lax.axis_index("s")

        def zrow(r, c2):
            for j in range(l // LANES):
                zeros_v[r, pl.ds(j * LANES, LANES)] = jnp.zeros((LANES,), F32)
            return c2

        lax.fori_loop(0, zr, zrow, 0)
        row0 = sid * rpt
        for k in range(rpt // zr):
            pltpu.sync_copy(zeros_v, acc.at[pl.ds(row0 + k * zr, zr)])
        plsc.subcore_barrier()

        wid = cid * NS + sid
        base0 = wid * nper
        pltpu.sync_copy(dsts_hbm.at[wid], idx_v)

        bufs = ((rows0, seml0, semsc0), (rows1, seml1, semsc1))

        def fire_load(c, b):
            rows, seml, _ = bufs[b]
            base = pl.multiple_of(base0 + c * _CG, 8)
            pltpu.async_copy(upd_hbm.at[pl.ds(base, _CG)], rows, seml)

        def wait_load(b):
            rows, seml, _ = bufs[b]
            pltpu.make_async_copy(
                upd_hbm.at[pl.ds(base0, _CG)], rows, seml).wait()

        def fire_scat(c, b):
            rows, _, semsc = bufs[b]
            pltpu.async_copy(rows, acc.at[idx_v.at[c]], semsc, add=True)

        def wait_scat(b):
            rows, _, semsc = bufs[b]
            pltpu.make_async_copy(rows, acc.at[idx_v.at[0]], semsc).wait()

        fire_load(0, 0)
        fire_load(1, 1)

        def body(i, carry):
            for b in range(2):
                wait_load(b)
                fire_scat(2 * i + b, b)
            for b in range(2):
                c = 2 * i + b
                wait_scat(b)

                @pl.when(c + 2 < nchunks)
                def _():
                    fire_load(c + 2, b)
            return carry

        lax.fori_loop(0, npairs, body, 0)

        if has_tail:
            wait_load(0)
            fire_scat(nchunks - 1, 0)
            wait_scat(0)
        plsc.subcore_barrier()

        def copy_out(out_ref):
            @pl.when(row0 + rpt <= n)
            def _():
                pltpu.sync_copy(acc.at[pl.ds(row0, rpt)],
                                out_ref.at[pl.ds(row0, rpt)])
            if tail_len > 0:
                @pl.when(row0 == tail_start)
                def _():
                    pltpu.sync_copy(acc.at[pl.ds(tail_start, tail_len)],
                                    out_ref.at[pl.ds(tail_start, tail_len)])

        @pl.when(cid == 0)
        def _():
            copy_out(o0)

        @pl.when(cid == 1)
        def _():
            copy_out(o1)

    return scatter_sum


def _unpack_perm(l):
    perm = []
    for j in range(l // (2 * LANES)):
        base = j * 2 * LANES
        perm += [base + 2 * i for i in range(LANES)]
        perm += [base + 2 * i + 1 for i in range(LANES)]
    return jnp.asarray(perm, jnp.int32)


def kernel(x, edge_attr, params, edge_index):
    n, l = x.shape
    e = edge_attr.shape[0]
    nper = e // NW
    src = edge_index[0].reshape(NW, nper // _CG, _CG)
    dst = edge_index[1].reshape(NW, nper // _CG, _CG)

    gather_add = _make_gather_add(n, e, l)
    scatter_sum = _make_scatter_sum(n, e, l)

    perm = _unpack_perm(l)
    p = q = None
    for li, layer in enumerate(params):
        ep, npar = layer['edge'], layer['node']
        wc = ep['W1'][2 * l:][:, perm]
        if p is None:
            p, q = _compute_pq(x, ep['W1'][:l], ep['W1'][l:2 * l])
        g = gather_add(p, q, dst, src)
        stats = _edge_pass1(g, edge_attr, wc)
        upd = _edge_pass2(g, edge_attr, wc, stats, ep['g1'][perm],
                          ep['be1'][perm], ep['W2'][perm, :], ep['b2'])

        a0, a1 = scatter_sum(upd, dst)

        hn, nstats = _node_pass1(x, a0, a1, npar['W1'][:l], npar['W1'][l:])
        if li + 1 < len(params):
            nep = params[li + 1]['edge']
            x, p, q = _node_pass2(hn, x, nstats, npar['g1'], npar['be1'],
                                  npar['W2'], npar['b2'],
                                  nep['W1'][:l], nep['W1'][l:2 * l])
        else:
            x = _node_pass2(hn, x, nstats, npar['g1'], npar['be1'],
                            npar['W2'], npar['b2'])
        edge_attr = upd

    return x, edge_attr


# speedup vs baseline: 1.3262x; 1.0650x over previous
import functools

import jax
import jax.numpy as jnp
from jax import lax
from jax.experimental import pallas as pl
from jax.experimental.pallas import tpu as pltpu
from jax.experimental.pallas import tpu_sc as plsc

F32 = jnp.float32
BN_EPS = 1e-5
NC, NS, LANES = 2, 16, 16
NW = NC * NS


def _block_rows(rows, cap=8192):
    best = 8
    for b in range(8, cap + 1, 8):
        if rows % b == 0:
            best = b
    return best


def _pq_body(x_ref, wa_ref, wb_ref, p_ref, q_ref):
    x = x_ref[...]
    p_ref[...] = jnp.dot(x, wa_ref[...],
                         preferred_element_type=F32).astype(jnp.bfloat16)
    q_ref[...] = jnp.dot(x, wb_ref[...],
                         preferred_element_type=F32).astype(jnp.bfloat16)


def _compute_pq(x, wa, wb):
    n, l = x.shape
    br = _block_rows(n)
    return pl.pallas_call(
        _pq_body,
        grid=(n // br,),
        in_specs=[
            pl.BlockSpec((br, l), lambda i: (i, 0)),
            pl.BlockSpec((l, l), lambda i: (0, 0)),
            pl.BlockSpec((l, l), lambda i: (0, 0)),
        ],
        out_specs=[
            pl.BlockSpec((br, l), lambda i: (i, 0)),
            pl.BlockSpec((br, l), lambda i: (i, 0)),
        ],
        out_shape=[jax.ShapeDtypeStruct((n, l), jnp.bfloat16)] * 2,
    )(x, wa, wb)


def _stats_accumulate(h, stats_ref):
    @pl.when(pl.program_id(0) == 0)
    def _():
        stats_ref[...] = jnp.zeros_like(stats_ref)

    stats_ref[0:1, :] += jnp.sum(h, axis=0, keepdims=True)
    stats_ref[1:2, :] += jnp.sum(h * h, axis=0, keepdims=True)


def _edge_pass1_body(g_ref, ea_ref, wc_ref, stats_ref):
    hb = g_ref[...].astype(F32) + jnp.dot(ea_ref[...], wc_ref[...],
                                          preferred_element_type=F32)
    _stats_accumulate(hb, stats_ref)


def _edge_pass1(g, ea, wc):
    e, l = g.shape
    br = _block_rows(e)
    return pl.pallas_call(
        _edge_pass1_body,
        grid=(e // br,),
        in_specs=[
            pl.BlockSpec((br, l), lambda i: (i, 0)),
            pl.BlockSpec((br, l), lambda i: (i, 0)),
            pl.BlockSpec((l, l), lambda i: (0, 0)),
        ],
        out_specs=pl.BlockSpec((2, l), lambda i: (0, 0)),
        out_shape=jax.ShapeDtypeStruct((2, l), F32),
        compiler_params=pltpu.CompilerParams(
            dimension_semantics=("arbitrary",)),
    )(g, ea, wc)


def _bn_scale_shift(stats_ref, count, g1_ref, be1_ref):
    mean = stats_ref[0:1, :] / count
    var = stats_ref[1:2, :] / count - mean * mean
    scale = g1_ref[...] * lax.rsqrt(var + BN_EPS)
    shift = be1_ref[...] - mean * scale
    return scale, shift


def _edge_pass2_body(g_ref, ea_ref, wc_ref, stats_ref, g1_ref, be1_ref,
                     w2_ref, b2_ref, out_ref, *, count):
    scale, shift = _bn_scale_shift(stats_ref, count, g1_ref, be1_ref)
    ea = ea_ref[...]
    hb = g_ref[...].astype(F32) + jnp.dot(ea, wc_ref[...],
                                          preferred_element_type=F32)
    hn = jnp.maximum(hb * scale + shift, 0.0)
    y = jnp.dot(hn, w2_ref[...], preferred_element_type=F32) + b2_ref[...]
    out_ref[...] = jnp.maximum(y, 0.0) + ea


def _edge_pass2(g, ea, wc, stats, g1, be1, w2, b2):
    e, l = g.shape
    br = _block_rows(e)
    row = pl.BlockSpec((br, l), lambda i: (i, 0))
    vec = pl.BlockSpec((1, l), lambda i: (0, 0))
    w = pl.BlockSpec((l, l), lambda i: (0, 0))
    return pl.pallas_call(
        functools.partial(_edge_pass2_body, count=float(e)),
        grid=(e // br,),
        in_specs=[row, row, w, pl.BlockSpec((2, l), lambda i: (0, 0)),
                  vec, vec, w, vec],
        out_specs=row,
        out_shape=jax.ShapeDtypeStruct((e, l), F32),
    )(g, ea, wc, stats, g1.reshape(1, l), be1.reshape(1, l),
      w2, b2.reshape(1, l))


def _node_pass1_body(x_ref, a0_ref, a1_ref, wa_ref, wb_ref,
                     h_ref, stats_ref):
    h = (jnp.dot(x_ref[...], wa_ref[...], preferred_element_type=F32)
         + jnp.dot(a0_ref[...] + a1_ref[...], wb_ref[...],
                   preferred_element_type=F32))
    h_ref[...] = h
    _stats_accumulate(h, stats_ref)


def _node_pass1(x, a0, a1, wa, wb):
    n, l = x.shape
    br = _block_rows(n)
    row = pl.BlockSpec((br, l), lambda i: (i, 0))
    w = pl.BlockSpec((l, l), lambda i: (0, 0))
    return pl.pallas_call(
        _node_pass1_body,
        grid=(n // br,),
        in_specs=[row, row, row, w, w],
        out_specs=[row, pl.BlockSpec((2, l), lambda i: (0, 0))],
        out_shape=[jax.ShapeDtypeStruct((n, l), F32),
                   jax.ShapeDtypeStruct((2, l), F32)],
        compiler_params=pltpu.CompilerParams(
            dimension_semantics=("arbitrary",)),
    )(x, a0, a1, wa, wb)


def _node_pass2_out(h_ref, res_ref, stats_ref, g1_ref, be1_ref, w2_ref,
                    b2_ref, count):
    scale, shift = _bn_scale_shift(stats_ref, count, g1_ref, be1_ref)
    hn = jnp.maximum(h_ref[...] * scale + shift, 0.0)
    y = jnp.dot(hn, w2_ref[...], preferred_element_type=F32) + b2_ref[...]
    return jnp.maximum(y, 0.0) + res_ref[...]


def _node_pass2_body(h_ref, res_ref, stats_ref, g1_ref, be1_ref, w2_ref,
                     b2_ref, out_ref, *, count):
    out_ref[...] = _node_pass2_out(h_ref, res_ref, stats_ref, g1_ref,
                                   be1_ref, w2_ref, b2_ref, count)


def _node_pass2_pq_body(h_ref, res_ref, stats_ref, g1_ref, be1_ref, w2_ref,
                        b2_ref, wan_ref, wbn_ref, out_ref, p_ref, q_ref, *,
                        count):
    xn = _node_pass2_out(h_ref, res_ref, stats_ref, g1_ref, be1_ref, w2_ref,
                         b2_ref, count)
    out_ref[...] = xn
    p_ref[...] = jnp.dot(xn, wan_ref[...],
                         preferred_element_type=F32).astype(jnp.bfloat16)
    q_ref[...] = jnp.dot(xn, wbn_ref[...],
                         preferred_element_type=F32).astype(jnp.bfloat16)


def _node_pass2(h, res, stats, g1, be1, w2, b2, wan=None, wbn=None):
    r, l = h.shape
    br = _block_rows(r)
    row = pl.BlockSpec((br, l), lambda i: (i, 0))
    vec = pl.BlockSpec((1, l), lambda i: (0, 0))
    w = pl.BlockSpec((l, l), lambda i: (0, 0))
    stat = pl.BlockSpec((2, l), lambda i: (0, 0))
    args = (h, res, stats, g1.reshape(1, l), be1.reshape(1, l), w2,
            b2.reshape(1, l))
    if wan is None:
        return pl.pallas_call(
            functools.partial(_node_pass2_body, count=float(r)),
            grid=(r // br,),
            in_specs=[row, row, stat, vec, vec, w, vec],
            out_specs=row,
            out_shape=jax.ShapeDtypeStruct((r, l), F32),
        )(*args)
    return pl.pallas_call(
        functools.partial(_node_pass2_pq_body, count=float(r)),
        grid=(r // br,),
        in_specs=[row, row, stat, vec, vec, w, vec, w, w],
        out_specs=[row, row, row],
        out_shape=[jax.ShapeDtypeStruct((r, l), F32),
                   jax.ShapeDtypeStruct((r, l), jnp.bfloat16),
                   jax.ShapeDtypeStruct((r, l), jnp.bfloat16)],
    )(*args, wan, wbn)


_CG = 80


def _make_gather_add(n, e, l):
    nper = e // NW
    lw = l // 2
    assert nper % _CG == 0 and nper % 16 == 0
    nchunks = nper // _CG
    npairs = nchunks // 2
    has_tail = nchunks % 2 == 1
    assert npairs >= 2
    assert _CG % 16 == 0
    mesh = plsc.VectorSubcoreMesh(core_axis_name="c", subcore_axis_name="s",
                                  num_cores=NC, num_subcores=NS)

    @functools.partial(
        pl.kernel,
        out_type=jax.ShapeDtypeStruct((e, l), F32),
        mesh=mesh,
        compiler_params=pltpu.CompilerParams(needs_layout_passes=False,
                                             use_tc_tiling_on_sc=False),
        scratch_types=[
            pltpu.VMEM((nchunks, _CG), jnp.int32),
            pltpu.VMEM((nchunks, _CG), jnp.int32),
            pltpu.VMEM((_CG, l), jnp.bfloat16),
            pltpu.VMEM((_CG, l), jnp.bfloat16),
            pltpu.VMEM((_CG, l), F32),
            pltpu.VMEM((_CG, l), jnp.bfloat16),
            pltpu.VMEM((_CG, l), jnp.bfloat16),
            pltpu.VMEM((_CG, l), F32),
        ] + [pltpu.SemaphoreType.DMA] * 4,
    )
    def gather_add(p_hbm, q_hbm, dsts_hbm, srcs_hbm, g_hbm,
                   idxd, idxs, prow0, qrow0, orow0, prow1, qrow1, orow1,
                   semg0, semg1, semw0, semw1):
        wid = lax.axis_index("s") * NC + lax.axis_index("c")
        base0 = wid * nper
        pltpu.sync_copy(dsts_hbm.at[wid], idxd)
        pltpu.sync_copy(srcs_hbm.at[wid], idxs)

        bufs = ((prow0, qrow0, orow0, semg0, semw0),
                (prow1, qrow1, orow1, semg1, semw1))

        def fire_gather(c, b):
            prow, qrow, _, semg, _ = bufs[b]
            pltpu.async_copy(p_hbm.at[idxd.at[c]], prow, semg)
            pltpu.async_copy(q_hbm.at[idxs.at[c]], qrow, semg)

        def wait_gather(b):
            prow, qrow, _, semg, _ = bufs[b]
            pltpu.make_async_copy(p_hbm.at[idxd.at[0]], prow, semg).wait()
            pltpu.make_async_copy(q_hbm.at[idxs.at[0]], qrow, semg).wait()

        def add_rows(b):
            prow, qrow, orow, _, _ = bufs[b]

            @plsc.parallel_loop(0, _CG, 2, unroll=2)
            def _(r):
                for rr in range(2):
                    for j in range(l // (2 * LANES)):
                        base = j * 2 * LANES
                        sl = pl.ds(base, 2 * LANES)
                        a, b2 = plsc.unpack(prow[r + rr, sl] + qrow[r + rr, sl],
                                            format=plsc.PackFormat.INTERLEAVED)
                        orow[r + rr, pl.ds(base, LANES)] = a
                        orow[r + rr, pl.ds(base + LANES, LANES)] = b2

        def fire_wb(c, b):
            _, _, orow, _, semw = bufs[b]
            base = pl.multiple_of(base0 + c * _CG, 16)
            pltpu.async_copy(orow, g_hbm.at[pl.ds(base, _CG)], semw)

        def wait_wb(b):
            _, _, orow, _, semw = bufs[b]
            pltpu.make_async_copy(
                orow, g_hbm.at[pl.ds(base0, _CG)], semw).wait()

        fire_gather(0, 0)
        fire_gather(1, 1)
        wait_gather(0)
        add_rows(0)
        fire_gather(2, 0)
        fire_wb(0, 0)
        wait_gather(1)
        add_rows(1)
        fire_gather(3, 1)
        fire_wb(1, 1)

        def body(i, carry):
            for b in range(2):
                c = 2 * i + b
                wait_gather(b)
                wait_wb(b)
                add_rows(b)

                @pl.when(c + 2 < nchunks)
                def _():
                    fire_gather(c + 2, b)

                fire_wb(c, b)
            return carry

        lax.fori_loop(1, npairs, body, 0)

        if has_tail:
            wait_gather(0)
            wait_wb(0)
            add_rows(0)
            fire_wb(nchunks - 1, 0)
        wait_wb(0)
        wait_wb(1)

    return gather_add


def _make_scatter_sum(n, e, l):
    nper = e // NW
    nchunks = nper // _CG
    npairs = nchunks // 2
    has_tail = nchunks % 2 == 1
    zr = 64
    npad = -(-n // (NS * 128)) * NS * 128
    rpt = npad // NS
    tail_start = (n // rpt) * rpt
    tail_len = n - tail_start
    assert n % 8 == 0
    mesh = plsc.VectorSubcoreMesh(core_axis_name="c", subcore_axis_name="s",
                                  num_cores=NC, num_subcores=NS)

    @functools.partial(
        pl.kernel,
        out_type=[jax.ShapeDtypeStruct((n, l), F32)] * NC,
        mesh=mesh,
        scratch_types=[
            pltpu.VMEM((nchunks, _CG), jnp.int32),
            pltpu.VMEM((_CG, l), F32),
            pltpu.VMEM((_CG, l), F32),
            pltpu.VMEM((zr, l), F32),
            pltpu.VMEM_SHARED((npad, l), F32),
        ] + [pltpu.SemaphoreType.DMA] * 4,
    )
    def scatter_sum(upd_hbm, dsts_hbm, o0, o1,
                    idx_v, rows0, rows1, zeros_v, acc,
                    seml0, seml1, semsc0, semsc1):
        cid = lax.axis_index("c")
        sid = lax.axis_index("s")

        def zrow(r, c2):
            for j in range(l // LANES):
                zeros_v[r, pl.ds(j * LANES, LANES)] = jnp.zeros((LANES,), F32)
            return c2

        lax.fori_loop(0, zr, zrow, 0)
        row0 = sid * rpt
        for k in range(rpt // zr):
            pltpu.sync_copy(zeros_v, acc.at[pl.ds(row0 + k * zr, zr)])
        plsc.subcore_barrier()

        wid = cid * NS + sid
        base0 = wid * nper
        pltpu.sync_copy(dsts_hbm.at[wid], idx_v)

        bufs = ((rows0, seml0, semsc0), (rows1, seml1, semsc1))

        def fire_load(c, b):
            rows, seml, _ = bufs[b]
            base = pl.multiple_of(base0 + c * _CG, 8)
            pltpu.async_copy(upd_hbm.at[pl.ds(base, _CG)], rows, seml)

        def wait_load(b):
            rows, seml, _ = bufs[b]
            pltpu.make_async_copy(
                upd_hbm.at[pl.ds(base0, _CG)], rows, seml).wait()

        def fire_scat(c, b):
            rows, _, semsc = bufs[b]
            pltpu.async_copy(rows, acc.at[idx_v.at[c]], semsc, add=True)

        def wait_scat(b):
            rows, _, semsc = bufs[b]
            pltpu.make_async_copy(rows, acc.at[idx_v.at[0]], semsc).wait()

        fire_load(0, 0)
        fire_load(1, 1)

        def body(i, carry):
            for b in range(2):
                wait_load(b)
                fire_scat(2 * i + b, b)
            for b in range(2):
                c = 2 * i + b
                wait_scat(b)

                @pl.when(c + 2 < nchunks)
                def _():
                    fire_load(c + 2, b)
            return carry

        lax.fori_loop(0, npairs, body, 0)

        if has_tail:
            wait_load(0)
            fire_scat(nchunks - 1, 0)
            wait_scat(0)
        plsc.subcore_barrier()

        def copy_out(out_ref):
            @pl.when(row0 + rpt <= n)
            def _():
                pltpu.sync_copy(acc.at[pl.ds(row0, rpt)],
                                out_ref.at[pl.ds(row0, rpt)])
            if tail_len > 0:
                @pl.when(row0 == tail_start)
                def _():
                    pltpu.sync_copy(acc.at[pl.ds(tail_start, tail_len)],
                                    out_ref.at[pl.ds(tail_start, tail_len)])

        @pl.when(cid == 0)
        def _():
            copy_out(o0)

        @pl.when(cid == 1)
        def _():
            copy_out(o1)

    return scatter_sum


def _unpack_perm(l):
    perm = []
    for j in range(l // (2 * LANES)):
        base = j * 2 * LANES
        perm += [base + 2 * i for i in range(LANES)]
        perm += [base + 2 * i + 1 for i in range(LANES)]
    return jnp.asarray(perm, jnp.int32)


def kernel(x, edge_attr, params, edge_index):
    n, l = x.shape
    e = edge_attr.shape[0]
    nper = e // NW
    src = edge_index[0].reshape(NW, nper // _CG, _CG)
    dst = edge_index[1].reshape(NW, nper // _CG, _CG)

    gather_add = _make_gather_add(n, e, l)
    scatter_sum = _make_scatter_sum(n, e, l)

    perm = _unpack_perm(l)
    p = q = None
    for li, layer in enumerate(params):
        ep, npar = layer['edge'], layer['node']
        wc = ep['W1'][2 * l:][:, perm]
        if p is None:
            p, q = _compute_pq(x, ep['W1'][:l], ep['W1'][l:2 * l])
        g = gather_add(p, q, dst, src)
        stats = _edge_pass1(g, edge_attr, wc)
        upd = _edge_pass2(g, edge_attr, wc, stats, ep['g1'][perm],
                          ep['be1'][perm], ep['W2'][perm, :], ep['b2'])

        a0, a1 = scatter_sum(upd, dst)

        hn, nstats = _node_pass1(x, a0, a1, npar['W1'][:l], npar['W1'][l:])
        if li + 1 < len(params):
            nep = params[li + 1]['edge']
            x, p, q = _node_pass2(hn, x, nstats, npar['g1'], npar['be1'],
                                  npar['W2'], npar['b2'],
                                  nep['W1'][:l], nep['W1'][l:2 * l])
        else:
            x = _node_pass2(hn, x, nstats, npar['g1'], npar['be1'],
                            npar['W2'], npar['b2'])
        edge_attr = upd

    return x, edge_attr
